# baseline (speedup 1.0000x reference)
import functools

import jax
import jax.numpy as jnp
from jax import lax
from jax.experimental import pallas as pl
from jax.experimental.pallas import tpu as pltpu
from jax.experimental.pallas import tpu_sc as plsc

N_NODES = 10000
HIDDEN = 128
AGENTS = 4
PENDING = 8
ACTION = 64
BATCH = 64
N_EDGES = 320000

NC, NS = 2, 16
NW = NC * NS
EPW = 10240
E_PAD = NW * EPW
NCHUNK = 80
SLOT = 64
NSLOT = EPW // SLOT
N_PAD = 10240
ROWS_PT = N_PAD // NS
BAP = BATCH * AGENTS * PENDING
GPW = BAP // NW

_MESH = plsc.VectorSubcoreMesh(core_axis_name="c", subcore_axis_name="s")
_F32 = jnp.float32


NBUF = 4
RCHUNK = (N_EDGES - (NW - 1) * EPW) // (2 * SLOT)
NPCHUNK = NCHUNK - RCHUNK


@functools.partial(
    pl.kernel,
    out_type=[
        jax.ShapeDtypeStruct((NC, BAP, HIDDEN), _F32),
        jax.ShapeDtypeStruct((BAP, HIDDEN), _F32),
    ],
    mesh=_MESH,
    scratch_types=[
        pltpu.VMEM((2, 2, 2 * SLOT), jnp.int32),
        pltpu.VMEM((NBUF, SLOT), jnp.int32),
        pltpu.VMEM((NBUF, SLOT), jnp.int32),
        pltpu.VMEM((NBUF, SLOT, HIDDEN), _F32),
        pltpu.VMEM((2 * GPW,), jnp.int32),
        pltpu.VMEM((GPW,), jnp.int32),
        pltpu.SemaphoreType.DMA,
        pltpu.SemaphoreType.DMA,
        pltpu.SemaphoreType.DMA,
        pltpu.SemaphoreType.DMA,
        pltpu.SemaphoreType.DMA,
        pltpu.SemaphoreType.DMA,
        pltpu.SemaphoreType.DMA,
        pltpu.SemaphoreType.DMA,
        pltpu.SemaphoreType.DMA,
        pltpu.SemaphoreType.DMA,
        pltpu.VMEM_SHARED((N_PAD, HIDDEN), _F32),
    ],
)
def _segment_sum_sc(ei_hbm, pad_hbm, x_hbm, cidx_hbm, accg_hbm, xg_hbm,
                    ei_v, soff_v, doff_v, rows_v, ca_v, cx_v,
                    sem0, sem1, sem2, sem3, ssem0, ssem1, ssem2, ssem3,
                    esem0, esem1, acc_sh):
    c = lax.axis_index("c")
    s = lax.axis_index("s")
    w = c * NS + s
    sems = (sem0, sem1, sem2, sem3)
    ssems = (ssem0, ssem1, ssem2, ssem3)
    esems = (esem0, esem1)

    @pl.loop(0, SLOT)
    def _zrow(i):
        @pl.loop(0, HIDDEN // 16)
        def _zlane(k):
            rows_v.at[0, i, pl.ds(k * 16, 16)][...] = jnp.zeros((16,), _F32)

    @pl.loop(0, ROWS_PT // SLOT)
    def _zcopy(k):
        pltpu.async_copy(rows_v.at[0],
                         acc_sh.at[pl.ds(s * ROWS_PT + k * SLOT, SLOT)], sem0)

    @pl.loop(0, ROWS_PT // SLOT)
    def _zdrain(k):
        pltpu.make_async_copy(
            rows_v.at[0], acc_sh.at[pl.ds(s * ROWS_PT + k * SLOT, SLOT)],
            sem0).wait()

    def _stage(cc, q):
        is_pad = (w == NW - 1) & (cc >= RCHUNK)

        @pl.when(is_pad)
        def _():
            pltpu.async_copy(pad_hbm.at[cc - RCHUNK], ei_v.at[q], esems[q])

        @pl.when(~is_pad)
        def _():
            pltpu.async_copy(
                ei_hbm.at[:, pl.ds(w * EPW + cc * 2 * SLOT, 2 * SLOT)],
                ei_v.at[q], esems[q])

    def _wait_stage(q):
        pltpu.make_async_copy(pad_hbm.at[0], ei_v.at[q], esems[q]).wait()

    def _unpack(t, b, q):
        h = (t % 2) * SLOT
        for k in range(SLOT // 16):
            soff_v.at[b, pl.ds(k * 16, 16)][...] = (
                ei_v[q, 0, pl.ds(h + k * 16, 16)])
            doff_v.at[b, pl.ds(k * 16, 16)][...] = (
                ei_v[q, 1, pl.ds(h + k * 16, 16)])

    def _gather(b):
        pltpu.async_copy(x_hbm.at[soff_v.at[b]], rows_v.at[b], sems[b])

    plsc.subcore_barrier()

    _stage(0, 0)
    _stage(1, 1)
    _wait_stage(0)
    for b in range(NBUF):
        if b == 2:
            _wait_stage(1)
        _unpack(b, b, b // 2)
        _gather(b)
    _stage(2, 0)
    _stage(3, 1)

    @pl.loop(0, NSLOT // NBUF)
    def _body(j):
        for b in range(NBUF):
            pltpu.make_async_copy(x_hbm.at[soff_v.at[b]], rows_v.at[b],
                                  sems[b]).wait()
            pltpu.async_copy(rows_v.at[b], acc_sh.at[doff_v.at[b]], ssems[b],
                             add=True)
        for b in range(NBUF):
            t = j * NBUF + b
            pltpu.make_async_copy(rows_v.at[b], acc_sh.at[doff_v.at[b]],
                                  ssems[b]).wait()

            @pl.when(t + NBUF < NSLOT)
            def _refill():
                if b in (0, 2):
                    _wait_stage(b // 2)
                _unpack(t + NBUF, b, b // 2)
                _gather(b)
                if b in (1, 3):
                    cc = 2 * j + 4 + b // 2

                    @pl.when(cc < NCHUNK)
                    def _():
                        _stage(cc, b // 2)

    plsc.subcore_barrier()

    pltpu.sync_copy(cidx_hbm.at[pl.ds(s * 2 * GPW, 2 * GPW)], ca_v)
    pltpu.sync_copy(acc_sh.at[ca_v.at[pl.ds(0, GPW)]], rows_v.at[0])
    pltpu.sync_copy(rows_v.at[0], accg_hbm.at[c, pl.ds(s * 2 * GPW, GPW)])
    pltpu.sync_copy(acc_sh.at[ca_v.at[pl.ds(GPW, GPW)]], rows_v.at[1])
    pltpu.sync_copy(rows_v.at[1],
                    accg_hbm.at[c, pl.ds(s * 2 * GPW + GPW, GPW)])

    pltpu.sync_copy(cidx_hbm.at[pl.ds(w * GPW, GPW)], cx_v)
    pltpu.sync_copy(x_hbm.at[cx_v], rows_v.at[2])
    pltpu.sync_copy(rows_v.at[2], xg_hbm.at[pl.ds(w * GPW, GPW)])


def _head_body(ag, xg, jt, jtb, wg, ws, q, w1, b1, w2, b2, il, al):
    u = ag[0, 0] + ag[1, 0]
    v = jnp.maximum(
        jnp.dot(u, wg[...], preferred_element_type=_F32)
        + jnp.dot(xg[0], ws[...], preferred_element_type=_F32), 0.0)
    ilv = jnp.sum(v * q[0], axis=-1)
    hid = jnp.maximum(
        jnp.dot(v, w1[0], preferred_element_type=_F32) + b1[0], 0.0)
    out = jnp.dot(hid, w2[0], preferred_element_type=_F32) + b2[0]
    mask = jt[0] >= 0
    il[...] = jnp.where(mask, ilv.reshape(1, BATCH * PENDING),
                        -1e9).reshape(1, 1, BATCH * PENDING)
    alm = jnp.where(jtb[0] >= 0, out, -1e9)
    al[...] = alm.reshape(BATCH, 1, PENDING, ACTION)


def kernel(x, edge_index, conn_idx, job_types, W_gnn, W_self, q_ptr, W1, b1, W2, b2):
    padi = jnp.arange(NPCHUNK * 2 * SLOT, dtype=jnp.int32)
    pad_tiles = jnp.stack(
        [(padi % 8192).reshape(NPCHUNK, 2 * SLOT),
         (N_NODES + padi % (N_PAD - N_NODES)).reshape(NPCHUNK, 2 * SLOT)],
        axis=1)

    cidx = jnp.transpose(conn_idx, (1, 0, 2)).reshape(BAP)
    accg, xg = _segment_sum_sc(edge_index, pad_tiles, x, cidx)

    bp = BATCH * PENDING
    ag4 = accg.reshape(NC, AGENTS, bp, HIDDEN)
    xg3 = xg.reshape(AGENTS, bp, HIDDEN)
    jt3 = jnp.transpose(job_types, (1, 0, 2)).reshape(AGENTS, 1, bp)
    jtb = jnp.broadcast_to(
        jnp.transpose(job_types, (1, 0, 2)).reshape(AGENTS, bp, 1),
        (AGENTS, bp, ACTION))
    q3 = q_ptr.reshape(AGENTS, 1, HIDDEN)
    b13 = b1.reshape(AGENTS, 1, HIDDEN)
    b23 = b2.reshape(AGENTS, 1, ACTION)

    il_t, alloc = pl.pallas_call(
        _head_body,
        grid=(AGENTS,),
        in_specs=[
            pl.BlockSpec((NC, 1, bp, HIDDEN), lambda a: (0, a, 0, 0)),
            pl.BlockSpec((1, bp, HIDDEN), lambda a: (a, 0, 0)),
            pl.BlockSpec((1, 1, bp), lambda a: (a, 0, 0)),
            pl.BlockSpec((1, bp, ACTION), lambda a: (a, 0, 0)),
            pl.BlockSpec((HIDDEN, HIDDEN), lambda a: (0, 0)),
            pl.BlockSpec((HIDDEN, HIDDEN), lambda a: (0, 0)),
            pl.BlockSpec((1, 1, HIDDEN), lambda a: (a, 0, 0)),
            pl.BlockSpec((1, HIDDEN, HIDDEN), lambda a: (a, 0, 0)),
            pl.BlockSpec((1, 1, HIDDEN), lambda a: (a, 0, 0)),
            pl.BlockSpec((1, HIDDEN, ACTION), lambda a: (a, 0, 0)),
            pl.BlockSpec((1, 1, ACTION), lambda a: (a, 0, 0)),
        ],
        out_specs=[
            pl.BlockSpec((1, 1, bp), lambda a: (a, 0, 0)),
            pl.BlockSpec((BATCH, 1, PENDING, ACTION), lambda a: (0, a, 0, 0)),
        ],
        out_shape=[
            jax.ShapeDtypeStruct((AGENTS, 1, bp), _F32),
            jax.ShapeDtypeStruct((BATCH, AGENTS, PENDING, ACTION), _F32),
        ],
    )(ag4, xg3, jt3, jtb, W_gnn, W_self, q3, W1, b13, W2, b23)

    index_logits = jnp.transpose(il_t.reshape(AGENTS, BATCH, PENDING), (1, 0, 2))
    return index_logits, alloc


# speedup vs baseline: 1.1793x; 1.1793x over previous
import functools

import jax
import jax.numpy as jnp
from jax import lax
from jax.experimental import pallas as pl
from jax.experimental.pallas import tpu as pltpu
from jax.experimental.pallas import tpu_sc as plsc

N_NODES = 10000
HIDDEN = 128
AGENTS = 4
PENDING = 8
ACTION = 64
BATCH = 64
N_EDGES = 320000

NC, NS = 2, 16
NW = NC * NS
EPW = 10240
E_PAD = NW * EPW
NCHUNK = 80
SLOT = 64
NSLOT = EPW // SLOT
N_PAD = 10240
ROWS_PT = N_PAD // NS
BAP = BATCH * AGENTS * PENDING
GPW = BAP // NW

_MESH = plsc.VectorSubcoreMesh(core_axis_name="c", subcore_axis_name="s")
_F32 = jnp.float32


NBUF = 4
RCHUNK = (N_EDGES - (NW - 1) * EPW) // (2 * SLOT)
NPCHUNK = NCHUNK - RCHUNK


@functools.partial(
    pl.kernel,
    out_type=[
        jax.ShapeDtypeStruct((NC, BAP, HIDDEN), _F32),
        jax.ShapeDtypeStruct((BAP, HIDDEN), _F32),
    ],
    mesh=_MESH,
    scratch_types=[
        pltpu.VMEM((2, 2, 2 * SLOT), jnp.int32),
        pltpu.VMEM((NBUF, SLOT), jnp.int32),
        pltpu.VMEM((NBUF, SLOT), jnp.int32),
        pltpu.VMEM((NBUF, SLOT, HIDDEN), _F32),
        pltpu.VMEM((2 * GPW,), jnp.int32),
        pltpu.VMEM((GPW,), jnp.int32),
        pltpu.SemaphoreType.DMA,
        pltpu.SemaphoreType.DMA,
        pltpu.SemaphoreType.DMA,
        pltpu.SemaphoreType.DMA,
        pltpu.SemaphoreType.DMA,
        pltpu.SemaphoreType.DMA,
        pltpu.SemaphoreType.DMA,
        pltpu.SemaphoreType.DMA,
        pltpu.SemaphoreType.DMA,
        pltpu.SemaphoreType.DMA,
        pltpu.VMEM_SHARED((N_PAD, HIDDEN), _F32),
    ],
)
def _segment_sum_sc(ei_hbm, pad_hbm, x_hbm, cidx_hbm, accg_hbm, xg_hbm,
                    ei_v, soff_v, doff_v, rows_v, ca_v, cx_v,
                    sem0, sem1, sem2, sem3, ssem0, ssem1, ssem2, ssem3,
                    esem0, esem1, acc_sh):
    c = lax.axis_index("c")
    s = lax.axis_index("s")
    w = c * NS + s
    sems = (sem0, sem1, sem2, sem3)
    ssems = (ssem0, ssem1, ssem2, ssem3)
    esems = (esem0, esem1)

    @pl.loop(0, SLOT)
    def _zrow(i):
        @pl.loop(0, HIDDEN // 16)
        def _zlane(k):
            rows_v.at[0, i, pl.ds(k * 16, 16)][...] = jnp.zeros((16,), _F32)

    @pl.loop(0, ROWS_PT // SLOT)
    def _zcopy(k):
        pltpu.async_copy(rows_v.at[0],
                         acc_sh.at[pl.ds(s * ROWS_PT + k * SLOT, SLOT)], sem0)

    @pl.loop(0, ROWS_PT // SLOT)
    def _zdrain(k):
        pltpu.make_async_copy(
            rows_v.at[0], acc_sh.at[pl.ds(s * ROWS_PT + k * SLOT, SLOT)],
            sem0).wait()

    def _stage(cc, q):
        is_pad = (w == NW - 1) & (cc >= RCHUNK)

        @pl.when(is_pad)
        def _():
            pltpu.async_copy(pad_hbm.at[cc - RCHUNK], ei_v.at[q], esems[q])

        @pl.when(~is_pad)
        def _():
            pltpu.async_copy(
                ei_hbm.at[:, pl.ds(w * EPW + cc * 2 * SLOT, 2 * SLOT)],
                ei_v.at[q], esems[q])

    def _wait_stage(q):
        pltpu.make_async_copy(pad_hbm.at[0], ei_v.at[q], esems[q]).wait()

    def _unpack(t, b, q):
        h = (t % 2) * SLOT
        for k in range(SLOT // 16):
            soff_v.at[b, pl.ds(k * 16, 16)][...] = (
                ei_v[q, 0, pl.ds(h + k * 16, 16)])
            doff_v.at[b, pl.ds(k * 16, 16)][...] = (
                ei_v[q, 1, pl.ds(h + k * 16, 16)])

    def _gather(b):
        pltpu.async_copy(x_hbm.at[soff_v.at[b]], rows_v.at[b], sems[b])

    plsc.subcore_barrier()

    _stage(0, 0)
    _stage(1, 1)
    _wait_stage(0)
    for b in range(NBUF):
        if b == 2:
            _wait_stage(1)
        _unpack(b, b, b // 2)
        _gather(b)
    _stage(2, 0)
    _stage(3, 1)

    @pl.loop(0, NSLOT // NBUF)
    def _body(j):
        for b in range(NBUF):
            t = j * NBUF + b
            pltpu.make_async_copy(x_hbm.at[soff_v.at[b]], rows_v.at[b],
                                  sems[b]).wait()
            pltpu.sync_copy(rows_v.at[b], acc_sh.at[doff_v.at[b]], add=True)

            @pl.when(t + NBUF < NSLOT)
            def _refill():
                if b in (0, 2):
                    _wait_stage(b // 2)
                _unpack(t + NBUF, b, b // 2)
                _gather(b)
                if b in (1, 3):
                    cc = 2 * j + 4 + b // 2

                    @pl.when(cc < NCHUNK)
                    def _():
                        _stage(cc, b // 2)

    plsc.subcore_barrier()

    pltpu.sync_copy(cidx_hbm.at[pl.ds(s * 2 * GPW, 2 * GPW)], ca_v)
    pltpu.sync_copy(acc_sh.at[ca_v.at[pl.ds(0, GPW)]], rows_v.at[0])
    pltpu.sync_copy(rows_v.at[0], accg_hbm.at[c, pl.ds(s * 2 * GPW, GPW)])
    pltpu.sync_copy(acc_sh.at[ca_v.at[pl.ds(GPW, GPW)]], rows_v.at[1])
    pltpu.sync_copy(rows_v.at[1],
                    accg_hbm.at[c, pl.ds(s * 2 * GPW + GPW, GPW)])

    pltpu.sync_copy(cidx_hbm.at[pl.ds(w * GPW, GPW)], cx_v)
    pltpu.sync_copy(x_hbm.at[cx_v], rows_v.at[2])
    pltpu.sync_copy(rows_v.at[2], xg_hbm.at[pl.ds(w * GPW, GPW)])


def _head_body(ag, xg, jt, jtb, wg, ws, q, w1, b1, w2, b2, il, al):
    u = ag[0, 0] + ag[1, 0]
    v = jnp.maximum(
        jnp.dot(u, wg[...], preferred_element_type=_F32)
        + jnp.dot(xg[0], ws[...], preferred_element_type=_F32), 0.0)
    ilv = jnp.sum(v * q[0], axis=-1)
    hid = jnp.maximum(
        jnp.dot(v, w1[0], preferred_element_type=_F32) + b1[0], 0.0)
    out = jnp.dot(hid, w2[0], preferred_element_type=_F32) + b2[0]
    mask = jt[0] >= 0
    il[...] = jnp.where(mask, ilv.reshape(1, BATCH * PENDING),
                        -1e9).reshape(1, 1, BATCH * PENDING)
    alm = jnp.where(jtb[0] >= 0, out, -1e9)
    al[...] = alm.reshape(BATCH, 1, PENDING, ACTION)


def kernel(x, edge_index, conn_idx, job_types, W_gnn, W_self, q_ptr, W1, b1, W2, b2):
    padi = jnp.arange(NPCHUNK * 2 * SLOT, dtype=jnp.int32)
    pad_tiles = jnp.stack(
        [(padi % 8192).reshape(NPCHUNK, 2 * SLOT),
         (N_NODES + padi % (N_PAD - N_NODES)).reshape(NPCHUNK, 2 * SLOT)],
        axis=1)

    cidx = jnp.transpose(conn_idx, (1, 0, 2)).reshape(BAP)
    accg, xg = _segment_sum_sc(edge_index, pad_tiles, x, cidx)

    bp = BATCH * PENDING
    ag4 = accg.reshape(NC, AGENTS, bp, HIDDEN)
    xg3 = xg.reshape(AGENTS, bp, HIDDEN)
    jt3 = jnp.transpose(job_types, (1, 0, 2)).reshape(AGENTS, 1, bp)
    jtb = jnp.broadcast_to(
        jnp.transpose(job_types, (1, 0, 2)).reshape(AGENTS, bp, 1),
        (AGENTS, bp, ACTION))
    q3 = q_ptr.reshape(AGENTS, 1, HIDDEN)
    b13 = b1.reshape(AGENTS, 1, HIDDEN)
    b23 = b2.reshape(AGENTS, 1, ACTION)

    il_t, alloc = pl.pallas_call(
        _head_body,
        grid=(AGENTS,),
        in_specs=[
            pl.BlockSpec((NC, 1, bp, HIDDEN), lambda a: (0, a, 0, 0)),
            pl.BlockSpec((1, bp, HIDDEN), lambda a: (a, 0, 0)),
            pl.BlockSpec((1, 1, bp), lambda a: (a, 0, 0)),
            pl.BlockSpec((1, bp, ACTION), lambda a: (a, 0, 0)),
            pl.BlockSpec((HIDDEN, HIDDEN), lambda a: (0, 0)),
            pl.BlockSpec((HIDDEN, HIDDEN), lambda a: (0, 0)),
            pl.BlockSpec((1, 1, HIDDEN), lambda a: (a, 0, 0)),
            pl.BlockSpec((1, HIDDEN, HIDDEN), lambda a: (a, 0, 0)),
            pl.BlockSpec((1, 1, HIDDEN), lambda a: (a, 0, 0)),
            pl.BlockSpec((1, HIDDEN, ACTION), lambda a: (a, 0, 0)),
            pl.BlockSpec((1, 1, ACTION), lambda a: (a, 0, 0)),
        ],
        out_specs=[
            pl.BlockSpec((1, 1, bp), lambda a: (a, 0, 0)),
            pl.BlockSpec((BATCH, 1, PENDING, ACTION), lambda a: (0, a, 0, 0)),
        ],
        out_shape=[
            jax.ShapeDtypeStruct((AGENTS, 1, bp), _F32),
            jax.ShapeDtypeStruct((BATCH, AGENTS, PENDING, ACTION), _F32),
        ],
    )(ag4, xg3, jt3, jtb, W_gnn, W_self, q3, W1, b13, W2, b23)

    index_logits = jnp.transpose(il_t.reshape(AGENTS, BATCH, PENDING), (1, 0, 2))
    return index_logits, alloc
